# baseline (speedup 1.0000x reference)
import functools

import jax
import jax.numpy as jnp
from jax import lax
from jax.experimental import pallas as pl
from jax.experimental.pallas import tpu as pltpu
from jax.experimental.pallas import tpu_sc as plsc

B = 16384
HR = 100
RANK = 16
DIMC = 4
NPAIR = HR * HR
ROWW = 256

NC = 2
NS = 16
NW = NC * NS
PER_W = B // NW
CH = 128
NCHUNK = PER_W // CH
L = 16

H1B = 50
TC_GRID = HR // H1B


def _pair_kernel(c_ref, e_ref, o_ref):
    e5 = e_ref[...]
    eA = e5[:, :RANK]
    eB = e5[:, RANK:]
    dn = (((1,), (1,)), ((), ()))
    for i in range(H1B):
        c = c_ref[i]
        zA = lax.dot_general(eA, c, dn, preferred_element_type=jnp.float32)
        zB = lax.dot_general(eB, c, dn, preferred_element_type=jnp.float32)
        o_ref[pl.ds(i * 2 * HR, 2 * HR), pl.ds(0, 64)] = zA
        o_ref[pl.ds(i * 2 * HR, 2 * HR), pl.ds(64, 64)] = zB


def _pair_table(c2, e5):
    return pl.pallas_call(
        _pair_kernel,
        grid=(TC_GRID,),
        in_specs=[
            pl.BlockSpec((H1B, 64, RANK), lambda i: (i, 0, 0)),
            pl.BlockSpec((2 * HR, 32), lambda i: (0, 0)),
        ],
        out_specs=pl.BlockSpec((H1B * 2 * HR, 128), lambda i: (i, 0)),
        out_shape=jax.ShapeDtypeStruct((NPAIR * 2, 128), jnp.float32),
    )(c2, e5)


_MESH = plsc.VectorSubcoreMesh(core_axis_name="c", subcore_axis_name="s")


@functools.partial(
    pl.kernel,
    out_type=jax.ShapeDtypeStruct((B, 64), jnp.float32),
    mesh=_MESH,
    compiler_params=pltpu.CompilerParams(needs_layout_passes=False),
    scratch_types=[
        pltpu.VMEM((PER_W,), jnp.int32),
        pltpu.VMEM((2, CH), jnp.int32),
        pltpu.VMEM((2, CH), jnp.int32),
        pltpu.VMEM((HR * 64,), jnp.float32),
        pltpu.VMEM((2, CH, 128), jnp.float32),
        pltpu.VMEM((2, CH, 128), jnp.float32),
        pltpu.VMEM((CH * 65,), jnp.float32),
        pltpu.VMEM((CH, 64), jnp.float32),
        pltpu.SemaphoreType.DMA,
        pltpu.SemaphoreType.DMA,
        pltpu.SemaphoreType.DMA,
    ],
)
def _sc_lookup(x_hbm, t_hbm, s_hbm, out_hbm, x_v, idxa_v, idxb_v, s_v, tra_v,
               trb_v, ob2_v, ob_v, sem0, sem1, sem2):
    wid = lax.axis_index("s") * NC + lax.axis_index("c")
    base = wid * PER_W
    scopy = pltpu.async_copy(s_hbm, s_v, sem2)
    pltpu.sync_copy(x_hbm.at[pl.ds(base, PER_W)], x_v)

    lanes = lax.iota(jnp.int32, L)
    c100 = jnp.full((L,), 100, jnp.int32)
    c10000 = jnp.full((L,), 10000, jnp.int32)
    sems = (sem0, sem1)

    def prep_fire(c, slot):
        cbase = c * CH
        for i in range(CH // L):
            xv = x_v[pl.ds(cbase + i * L, L)]
            h1 = lax.rem(lax.div(xv, c100), c100)
            h2 = lax.div(xv, c10000)
            r2 = (h1 * 100 + h2) * 2
            idxa_v[slot, pl.ds(i * L, L)] = r2
            idxb_v[slot, pl.ds(i * L, L)] = r2 + 1
        pltpu.async_copy(t_hbm.at[idxa_v.at[slot]], tra_v.at[slot], sems[slot])
        pltpu.async_copy(t_hbm.at[idxb_v.at[slot]], trb_v.at[slot], sems[slot])

    def wait_slot(slot):
        pltpu.make_async_copy(t_hbm.at[idxa_v.at[slot]], tra_v.at[slot],
                              sems[slot]).wait()
        pltpu.make_async_copy(t_hbm.at[idxb_v.at[slot]], trb_v.at[slot],
                              sems[slot]).wait()

    def compute(c, slot):
        cbase = c * CH
        tra = tra_v.at[slot]
        trb = trb_v.at[slot]

        def group_body(g, carry2):
            xg = x_v[pl.ds(cbase + g * L, L)]
            sbase = lax.rem(xg, c100) * 64
            rowg = g * L + lanes
            obase = (g * L + lanes) * 65
            sb = [sbase + D * 16 for D in range(DIMC)]
            zero = jnp.zeros((L,), jnp.float32)
            for d in range(DIMC):
                def t_body(t, accs):
                    rot = (lanes + t) & 15
                    svals = [plsc.load_gather(s_v, [sb[D] + rot])
                             for D in range(DIMC)]
                    new = list(accs)
                    for e in range(DIMC):
                        tref = tra if e < 2 else trb
                        tval = plsc.load_gather(
                            tref, [rowg, ((e % 2) * 64 + d * 16) + rot])
                        for D in range(DIMC):
                            new[D * 4 + e] = new[D * 4 + e] + svals[D] * tval
                    return tuple(new)
                accs = lax.fori_loop(0, RANK, t_body, (zero,) * 16)
                for D in range(DIMC):
                    for e in range(DIMC):
                        j = D * 16 + d * 4 + e
                        plsc.store_scatter(ob2_v, [obase + j], accs[D * 4 + e])
            return carry2
        lax.fori_loop(0, CH // L, group_body, 0)

        def comp_body(i, carry2):
            for r in range(4):
                b = i * 4 + r
                for k in range(4):
                    ob_v[b, pl.ds(k * L, L)] = ob2_v[pl.ds(b * 65 + k * L, L)]
            return carry2
        lax.fori_loop(0, CH // 4, comp_body, 0)

        pltpu.sync_copy(ob_v, out_hbm.at[pl.ds(base + cbase, CH)])

    prep_fire(0, 0)
    scopy.wait()

    def outer(k, carry):
        c0 = k * 2
        prep_fire(c0 + 1, 1)
        wait_slot(0)
        compute(c0, 0)

        @pl.when(k + 1 < NCHUNK // 2)
        def _():
            prep_fire(c0 + 2, 0)
        wait_slot(1)
        compute(c0 + 1, 1)
        return carry

    lax.fori_loop(0, NCHUNK // 2, outer, 0)


def kernel(x, start_core, end_core, cores):
    c2 = cores[0].reshape(HR, 64, RANK)
    e5 = end_core.reshape(2 * HR, 32)
    t = _pair_table(c2, e5)
    s_flat = start_core.reshape(HR * 64)
    return _sc_lookup(x.astype(jnp.int32), t, s_flat)


# speedup vs baseline: 1.0402x; 1.0402x over previous
import functools

import jax
import jax.numpy as jnp
from jax import lax
from jax.experimental import pallas as pl
from jax.experimental.pallas import tpu as pltpu
from jax.experimental.pallas import tpu_sc as plsc

B = 16384
HR = 100
RANK = 16
DIMC = 4
NPAIR = HR * HR
ROWW = 256

NC = 2
NS = 16
NW = NC * NS
PER_W = B // NW
CH = 128
NCHUNK = PER_W // CH
L = 16

H1B = 50
TC_GRID = HR // H1B


def _pair_kernel(bd_ref, e_ref, o_ref):
    e5 = e_ref[...]
    for i in range(H1B):
        z = jnp.dot(e5, bd_ref[i], preferred_element_type=jnp.float32)
        o_ref[pl.ds(i * 2 * HR, 2 * HR), :] = z


def _pair_table(bd, e5):
    return pl.pallas_call(
        _pair_kernel,
        grid=(TC_GRID,),
        in_specs=[
            pl.BlockSpec((H1B, 32, 128), lambda i: (i, 0, 0)),
            pl.BlockSpec((2 * HR, 32), lambda i: (0, 0)),
        ],
        out_specs=pl.BlockSpec((H1B * 2 * HR, 128), lambda i: (i, 0)),
        out_shape=jax.ShapeDtypeStruct((NPAIR * 2, 128), jnp.float32),
    )(bd, e5)


_MESH = plsc.VectorSubcoreMesh(core_axis_name="c", subcore_axis_name="s")


@functools.partial(
    pl.kernel,
    out_type=jax.ShapeDtypeStruct((B, 64), jnp.float32),
    mesh=_MESH,
    compiler_params=pltpu.CompilerParams(needs_layout_passes=False),
    scratch_types=[
        pltpu.VMEM((PER_W,), jnp.int32),
        pltpu.VMEM((2, CH), jnp.int32),
        pltpu.VMEM((2, CH), jnp.int32),
        pltpu.VMEM((HR * 64,), jnp.float32),
        pltpu.VMEM((2, CH, 128), jnp.float32),
        pltpu.VMEM((2, CH, 128), jnp.float32),
        pltpu.VMEM((CH * 65,), jnp.float32),
        pltpu.VMEM((CH, 64), jnp.float32),
        pltpu.SemaphoreType.DMA,
        pltpu.SemaphoreType.DMA,
        pltpu.SemaphoreType.DMA,
    ],
)
def _sc_lookup(x_hbm, t_hbm, s_hbm, out_hbm, x_v, idxa_v, idxb_v, s_v, tra_v,
               trb_v, ob2_v, ob_v, sem0, sem1, sem2):
    wid = lax.axis_index("s") * NC + lax.axis_index("c")
    base = wid * PER_W
    scopy = pltpu.async_copy(s_hbm, s_v, sem2)
    pltpu.sync_copy(x_hbm.at[pl.ds(base, PER_W)], x_v)

    lanes = lax.iota(jnp.int32, L)
    c100 = jnp.full((L,), 100, jnp.int32)
    c10000 = jnp.full((L,), 10000, jnp.int32)
    sems = (sem0, sem1)

    def prep_fire(c, slot):
        cbase = c * CH
        for i in range(CH // L):
            xv = x_v[pl.ds(cbase + i * L, L)]
            h1 = lax.rem(lax.div(xv, c100), c100)
            h2 = lax.div(xv, c10000)
            r2 = (h1 * 100 + h2) * 2
            idxa_v[slot, pl.ds(i * L, L)] = r2
            idxb_v[slot, pl.ds(i * L, L)] = r2 + 1
        pltpu.async_copy(t_hbm.at[idxa_v.at[slot]], tra_v.at[slot], sems[slot])
        pltpu.async_copy(t_hbm.at[idxb_v.at[slot]], trb_v.at[slot], sems[slot])

    def wait_slot(slot):
        pltpu.make_async_copy(t_hbm.at[idxa_v.at[slot]], tra_v.at[slot],
                              sems[slot]).wait()
        pltpu.make_async_copy(t_hbm.at[idxb_v.at[slot]], trb_v.at[slot],
                              sems[slot]).wait()

    def compute(c, slot):
        cbase = c * CH
        tra = tra_v.at[slot]
        trb = trb_v.at[slot]

        def group_body(g, carry2):
            xg = x_v[pl.ds(cbase + g * L, L)]
            sbase = lax.rem(xg, c100) * 64
            rowg = g * L + lanes
            obase = (g * L + lanes) * 65
            sb = [sbase + D * 16 for D in range(DIMC)]
            zero = jnp.zeros((L,), jnp.float32)
            for d in range(DIMC):
                def t_body(t, accs):
                    rot = (lanes + t) & 15
                    svals = [plsc.load_gather(s_v, [sb[D] + rot])
                             for D in range(DIMC)]
                    new = list(accs)
                    for e in range(DIMC):
                        tref = tra if e < 2 else trb
                        tval = plsc.load_gather(
                            tref, [rowg, ((e % 2) * 64 + d * 16) + rot])
                        for D in range(DIMC):
                            new[D * 4 + e] = new[D * 4 + e] + svals[D] * tval
                    return tuple(new)
                accs = lax.fori_loop(0, RANK, t_body, (zero,) * 16)
                for D in range(DIMC):
                    for e in range(DIMC):
                        j = D * 16 + d * 4 + e
                        plsc.store_scatter(ob2_v, [obase + j], accs[D * 4 + e])
            return carry2
        lax.fori_loop(0, CH // L, group_body, 0)

        def comp_body(i, carry2):
            for r in range(4):
                b = i * 4 + r
                for k in range(4):
                    ob_v[b, pl.ds(k * L, L)] = ob2_v[pl.ds(b * 65 + k * L, L)]
            return carry2
        lax.fori_loop(0, CH // 4, comp_body, 0)

        pltpu.sync_copy(ob_v, out_hbm.at[pl.ds(base + cbase, CH)])

    prep_fire(0, 0)
    scopy.wait()

    def outer(k, carry):
        c0 = k * 2
        prep_fire(c0 + 1, 1)
        wait_slot(0)
        compute(c0, 0)

        @pl.when(k + 1 < NCHUNK // 2)
        def _():
            prep_fire(c0 + 2, 0)
        wait_slot(1)
        compute(c0 + 1, 1)
        return carry

    lax.fori_loop(0, NCHUNK // 2, outer, 0)


def kernel(x, start_core, end_core, cores):
    ct = jnp.transpose(cores[0].reshape(HR, 64, RANK), (0, 2, 1))
    bd = (jnp.zeros((HR, 2, RANK, 2, 64), jnp.float32)
          .at[:, 0, :, 0, :].set(ct)
          .at[:, 1, :, 1, :].set(ct)
          .reshape(HR, 32, 128))
    e5 = end_core.reshape(2 * HR, 32)
    t = _pair_table(bd, e5)
    s_flat = start_core.reshape(HR * 64)
    return _sc_lookup(x.astype(jnp.int32), t, s_flat)
